# baseline (speedup 1.0000x reference)
import functools

import jax
import jax.numpy as jnp
from jax import lax
from jax.experimental import pallas as pl
from jax.experimental.pallas import tpu as pltpu

TB = 1280
NBLK = 8
NPAD = TB * NBLK
CW = 2 * TB
NCH = NBLK // 2


def _chunkA_of(a, j):
    cd = a // 2
    last = j == NCH - 1
    c = j + (j >= cd).astype(jnp.int32)
    return jnp.where(last, cd, c)


def _passA_body(n, L_ref, xf_ref, y1_ref, z_ref, y1acc_ref, ay_ref, az_ref):
    a = pl.program_id(0)
    j = pl.program_id(1)
    c = _chunkA_of(a, j)
    edge = c == NCH - 1
    ntail = n - (NCH - 1) * CW

    @pl.when(j == 0)
    def _init():
        ay_ref[...] = jnp.zeros_like(ay_ref)
        az_ref[...] = jnp.zeros_like(az_ref)

    def _work(Lc):
        ay_ref[...] += jnp.dot(Lc, xf_ref[c],
                               preferred_element_type=jnp.float32)
        s0 = 2 * c
        s1 = 2 * c + 1

        @pl.when(s0 < a)
        def _lo0():
            az_ref[...] += jnp.dot(Lc[:, :TB], y1acc_ref[s0],
                                   preferred_element_type=jnp.float32)

        @pl.when(s1 < a)
        def _lo1():
            az_ref[...] += jnp.dot(Lc[:, TB:], y1acc_ref[s1],
                                   preferred_element_type=jnp.float32)

        @pl.when(j == NCH - 1)
        def _finalize():
            row = lax.broadcasted_iota(jnp.int32, (TB, 1), 0)
            y1_a = ay_ref[...]
            y1_a = jnp.where(
                jnp.logical_or(a < NBLK - 1, row < n - (NBLK - 1) * TB),
                y1_a, 0.0)
            y1acc_ref[a] = y1_a
            y1_ref[0] = y1_a

            @pl.when(a % 2 == 0)
            def _diag_lo():
                z_ref[0] = az_ref[...] + jnp.dot(
                    Lc[:, :TB], y1_a, preferred_element_type=jnp.float32)

            @pl.when(a % 2 == 1)
            def _diag_hi():
                z_ref[0] = az_ref[...] + jnp.dot(
                    Lc[:, TB:], y1_a, preferred_element_type=jnp.float32)

    @pl.when(jnp.logical_not(edge))
    def _body():
        _work(L_ref[...])

    @pl.when(edge)
    def _body_edge():
        col = lax.broadcasted_iota(jnp.int32, (TB, CW), 1)
        _work(jnp.where(col < ntail, L_ref[...], 0.0))


def _passB_body(n, L_ref, y1_ref, z_ref, xf_ref, w_ref, b_ref, out_ref,
                acc_ref):
    a = pl.program_id(0)
    j = pl.program_id(1)
    jmin = a + 1
    edge = j == NBLK - 1
    ntail = n - (NBLK - 1) * TB

    @pl.when(j == 0)
    def _load():
        acc_ref[...] = z_ref[0]

    @pl.when(jnp.logical_and(j >= jmin, jnp.logical_not(edge)))
    def _upper():
        acc_ref[...] += jnp.dot(L_ref[...], y1_ref[j],
                                preferred_element_type=jnp.float32)

    @pl.when(edge)
    def _edge_and_emit():
        @pl.when(j >= jmin)
        def _upper_edge():
            col = lax.broadcasted_iota(jnp.int32, (TB, TB), 1)
            Lc = jnp.where(col < ntail, L_ref[...], 0.0)
            acc_ref[...] += jnp.dot(Lc, y1_ref[j],
                                    preferred_element_type=jnp.float32)

        w0 = w_ref[0]
        w1 = w_ref[1]
        w2 = w_ref[2]
        out = jnp.dot(xf_ref[a], w0 - w2, preferred_element_type=jnp.float32)
        out += jnp.dot(y1_ref[a], w1, preferred_element_type=jnp.float32)
        out += jnp.dot(2.0 * acc_ref[...], w2,
                       preferred_element_type=jnp.float32)
        out_ref[...] = out + b_ref[...]


@jax.jit
def kernel(x, L_cheb, weight, bias):
    tasks, n, c = x.shape
    kdeg = weight.shape[1]
    tc = tasks * c

    xf = jnp.transpose(x, (1, 0, 2)).reshape(n, tc)
    xfp = jnp.zeros((NPAD, tc), jnp.float32).at[:n].set(xf)
    xf3 = xfp.reshape(NBLK, TB, tc)
    xfc = xfp.reshape(NCH, CW, tc)
    eye = jnp.eye(tasks, dtype=weight.dtype)
    wbd = jnp.einsum('ts,tkio->ksito', eye, weight).reshape(
        kdeg, tasks * c, tasks * weight.shape[-1])
    bias_flat = bias.reshape(1, tasks * bias.shape[-1])

    y13, z3 = pl.pallas_call(
        functools.partial(_passA_body, n),
        grid=(NBLK, NCH),
        in_specs=[
            pl.BlockSpec((TB, CW), lambda a, j: (a, _chunkA_of(a, j))),
            pl.BlockSpec((NCH, CW, tc), lambda a, j: (0, 0, 0)),
        ],
        out_specs=[
            pl.BlockSpec((1, TB, tc), lambda a, j: (a, 0, 0)),
            pl.BlockSpec((1, TB, tc), lambda a, j: (a, 0, 0)),
        ],
        out_shape=[
            jax.ShapeDtypeStruct((NBLK, TB, tc), jnp.float32),
            jax.ShapeDtypeStruct((NBLK, TB, tc), jnp.float32),
        ],
        scratch_shapes=[
            pltpu.VMEM((NBLK, TB, tc), jnp.float32),
            pltpu.VMEM((TB, tc), jnp.float32),
            pltpu.VMEM((TB, tc), jnp.float32),
        ],
    )(L_cheb, xfc)

    out_f = pl.pallas_call(
        functools.partial(_passB_body, n),
        grid=(NBLK, NBLK),
        in_specs=[
            pl.BlockSpec(
                (TB, TB),
                lambda a, j: (a, jnp.minimum(jnp.maximum(j, a + 1),
                                             NBLK - 1))),
            pl.BlockSpec((NBLK, TB, tc), lambda a, j: (0, 0, 0)),
            pl.BlockSpec((1, TB, tc), lambda a, j: (a, 0, 0)),
            pl.BlockSpec((NBLK, TB, tc), lambda a, j: (0, 0, 0)),
            pl.BlockSpec(wbd.shape, lambda a, j: (0, 0, 0)),
            pl.BlockSpec((1, tc), lambda a, j: (0, 0)),
        ],
        out_specs=pl.BlockSpec((TB, tc), lambda a, j: (a, 0)),
        out_shape=jax.ShapeDtypeStruct((n, tc), jnp.float32),
        scratch_shapes=[pltpu.VMEM((TB, tc), jnp.float32)],
    )(L_cheb, y13, z3, xf3, wbd, bias_flat)

    return jnp.transpose(out_f.reshape(n, tasks, c), (1, 0, 2))
